# Initial kernel scaffold; baseline (speedup 1.0000x reference)
import jax
import jax.numpy as jnp
from jax.experimental import pallas as pl


def kernel(x, t, noise, W1, b1, W2, b2):
    raise NotImplementedError("write your pallas kernel here")


# speedup vs baseline: 2.1378x; 2.1378x over previous
import numpy as np
import jax
import jax.numpy as jnp
from jax.experimental import pallas as pl
from jax.experimental.pallas import tpu as pltpu

_SIGMA1 = 0.02
_K = 128
_TMIN = 1e-10
_LEAKY = 0.01
_LN_S1 = float(np.log(_SIGMA1))

_B_BLK = 128
_C_BLK = 128


def _fused_kernel(x_ref, t_ref, noise_ref, W1_ref, b1_ref, W2_ref, b2_ref,
                  out_ref, mu_scr, h_scr, o_scr):
    D = x_ref.shape[1]
    t = t_ref[...]
    gamma = 1.0 - jnp.exp((2.0 * _LN_S1) * t)
    mu_coef = gamma * (1.0 - gamma)
    mu_scr[...] = gamma * x_ref[...] + mu_coef * noise_ref[...]

    h = jnp.dot(mu_scr[...], W1_ref[:D, :], preferred_element_type=jnp.float32)
    h = h + t * W1_ref[D:D + 1, :] + b1_ref[...]
    h_scr[...] = jnp.where(h >= 0, h, _LEAKY * h)

    o_scr[...] = jnp.dot(h_scr[...], W2_ref[...], preferred_element_type=jnp.float32)

    inv_gamma = 1.0 / gamma
    r = jnp.sqrt((1.0 - gamma) * inv_gamma)
    low_t = t < _TMIN
    w_row = jnp.exp((-2.0 * _LN_S1) * t)

    n_chunks = D // _C_BLK
    part = jnp.zeros((_B_BLK, _C_BLK), jnp.float32)
    for c in range(n_chunks):
        lo, hi = c * _C_BLK, (c + 1) * _C_BLK
        mu_eps = o_scr[:, lo:hi] + b2_ref[:, lo:hi]
        ln_sig = o_scr[:, D + lo:D + hi] + b2_ref[:, D + lo:D + hi]
        mu_c = mu_scr[:, lo:hi]
        mu_x = mu_c * inv_gamma - r * mu_eps
        sigma_x = r * jnp.exp(ln_sig)
        mu_x = jnp.where(low_t, 0.0, mu_x)
        sigma_x = jnp.where(low_t, 1.0, sigma_x)
        inv = 1.0 / (sigma_x * jnp.float32(np.sqrt(2.0)))

        def body(k, acc):
            b_k = (2.0 / _K) * k.astype(jnp.float32) - 1.0
            return acc + jax.lax.erf((b_k - mu_x) * inv)

        s = jax.lax.fori_loop(1, _K - 1, body, jnp.zeros_like(mu_x))
        e_last = jax.lax.erf(((2.0 * (_K - 1) / _K - 1.0) - mu_x) * inv)
        pO = (-127.0 / 256.0) + (125.0 / 256.0) * e_last - (1.0 / 128.0) * s
        d = x_ref[:, lo:hi] - pO
        part = part + w_row * (d * d)

    out_ref[...] = jnp.sum(part, axis=0, keepdims=True).reshape(1, 1, _C_BLK)


def kernel(x, t, noise, W1, b1, W2, b2):
    B, D = x.shape
    H = W1.shape[1]
    nb = B // _B_BLK
    grid = (nb,)
    parts = pl.pallas_call(
        _fused_kernel,
        grid=grid,
        in_specs=[
            pl.BlockSpec((_B_BLK, D), lambda i: (i, 0)),
            pl.BlockSpec((_B_BLK, 1), lambda i: (i, 0)),
            pl.BlockSpec((_B_BLK, D), lambda i: (i, 0)),
            pl.BlockSpec((D + 1, H), lambda i: (0, 0)),
            pl.BlockSpec((1, H), lambda i: (0, 0)),
            pl.BlockSpec((H, 2 * D), lambda i: (0, 0)),
            pl.BlockSpec((1, 2 * D), lambda i: (0, 0)),
        ],
        out_specs=pl.BlockSpec((1, 1, _C_BLK), lambda i: (i, 0, 0)),
        out_shape=jax.ShapeDtypeStruct((nb, 1, _C_BLK), jnp.float32),
        scratch_shapes=[
            pltpu.VMEM((_B_BLK, D), jnp.float32),
            pltpu.VMEM((_B_BLK, H), jnp.float32),
            pltpu.VMEM((_B_BLK, 2 * D), jnp.float32),
        ],
        compiler_params=pltpu.CompilerParams(
            dimension_semantics=("parallel",),
            vmem_limit_bytes=100 * 1024 * 1024,
        ),
    )(x, t, noise, W1, b1.reshape(1, H), W2, b2.reshape(1, 2 * D))
    return (-_LN_S1 / (B * D)) * jnp.sum(parts)


# speedup vs baseline: 3.0121x; 1.4090x over previous
import numpy as np
import jax
import jax.numpy as jnp
from jax.experimental import pallas as pl
from jax.experimental.pallas import tpu as pltpu

_SIGMA1 = 0.02
_K = 128
_TMIN = 1e-10
_LEAKY = 0.01
_LN_S1 = float(np.log(_SIGMA1))

_B_BLK = 128
_C_BLK = 256


def _fused_kernel(x_ref, t_ref, noise_ref, W1_ref, b1_ref, W2_ref, b2_ref,
                  out_ref, mu_scr, h_scr, o_scr):
    D = x_ref.shape[1]
    t = t_ref[...]
    gamma = 1.0 - jnp.exp((2.0 * _LN_S1) * t)
    mu_coef = gamma * (1.0 - gamma)
    mu_scr[...] = gamma * x_ref[...] + mu_coef * noise_ref[...]

    h = jnp.dot(mu_scr[...], W1_ref[:D, :], preferred_element_type=jnp.float32)
    h = h + t * W1_ref[D:D + 1, :] + b1_ref[...]
    h_scr[...] = jnp.where(h >= 0, h, _LEAKY * h)

    o_scr[...] = jnp.dot(h_scr[...], W2_ref[...], preferred_element_type=jnp.float32)

    inv_gamma = 1.0 / gamma
    r = jnp.sqrt((1.0 - gamma) * inv_gamma)
    low_t = t < _TMIN
    w_row = jnp.exp((-2.0 * _LN_S1) * t)

    n_chunks = D // _C_BLK
    part = jnp.zeros((_B_BLK, _C_BLK), jnp.float32)
    for c in range(n_chunks):
        lo, hi = c * _C_BLK, (c + 1) * _C_BLK
        mu_eps = o_scr[:, lo:hi] + b2_ref[:, lo:hi]
        ln_sig = o_scr[:, D + lo:D + hi] + b2_ref[:, D + lo:D + hi]
        mu_c = mu_scr[:, lo:hi]
        mu_x = mu_c * inv_gamma - r * mu_eps
        sigma_x = r * jnp.exp(ln_sig)
        mu_x = jnp.where(low_t, 0.0, mu_x)
        sigma_x = jnp.where(low_t, 1.0, sigma_x)
        inv = jnp.minimum(1.0 / (sigma_x * jnp.float32(np.sqrt(2.0))), 1e30)
        ds = (2.0 / _K) * inv
        z1 = ((2.0 / _K - 1.0) - mu_x) * inv

        def body(k, carry):
            z, a0, a1 = carry
            e0 = jax.lax.erf(z)
            z = z + ds
            e1 = jax.lax.erf(z)
            z = z + ds
            return z, a0 + e0, a1 + e1

        zf, a0, a1 = jax.lax.fori_loop(
            0, (_K - 2) // 2, body,
            (z1, jnp.zeros_like(mu_x), jnp.zeros_like(mu_x)), unroll=3)
        s = a0 + a1
        e_last = jax.lax.erf(zf)
        pO = (-127.0 / 256.0) + (125.0 / 256.0) * e_last - (1.0 / 128.0) * s
        d = x_ref[:, lo:hi] - pO
        part = part + w_row * (d * d)

    out_ref[...] = jnp.sum(part, axis=0, keepdims=True).reshape(1, 1, _C_BLK)


def kernel(x, t, noise, W1, b1, W2, b2):
    B, D = x.shape
    H = W1.shape[1]
    nb = B // _B_BLK
    grid = (nb,)
    parts = pl.pallas_call(
        _fused_kernel,
        grid=grid,
        in_specs=[
            pl.BlockSpec((_B_BLK, D), lambda i: (i, 0)),
            pl.BlockSpec((_B_BLK, 1), lambda i: (i, 0)),
            pl.BlockSpec((_B_BLK, D), lambda i: (i, 0)),
            pl.BlockSpec((D + 1, H), lambda i: (0, 0)),
            pl.BlockSpec((1, H), lambda i: (0, 0)),
            pl.BlockSpec((H, 2 * D), lambda i: (0, 0)),
            pl.BlockSpec((1, 2 * D), lambda i: (0, 0)),
        ],
        out_specs=pl.BlockSpec((1, 1, _C_BLK), lambda i: (i, 0, 0)),
        out_shape=jax.ShapeDtypeStruct((nb, 1, _C_BLK), jnp.float32),
        scratch_shapes=[
            pltpu.VMEM((_B_BLK, D), jnp.float32),
            pltpu.VMEM((_B_BLK, H), jnp.float32),
            pltpu.VMEM((_B_BLK, 2 * D), jnp.float32),
        ],
        compiler_params=pltpu.CompilerParams(
            dimension_semantics=("parallel",),
            vmem_limit_bytes=100 * 1024 * 1024,
        ),
    )(x, t, noise, W1, b1.reshape(1, H), W2, b2.reshape(1, 2 * D))
    return (-_LN_S1 / (B * D)) * jnp.sum(parts)


# speedup vs baseline: 3.3033x; 1.0967x over previous
import numpy as np
import jax
import jax.numpy as jnp
from jax.experimental import pallas as pl
from jax.experimental.pallas import tpu as pltpu

_SIGMA1 = 0.02
_K = 128
_TMIN = 1e-10
_LEAKY = 0.01
_LN_S1 = float(np.log(_SIGMA1))

_B_BLK = 128
_C_BLK = 128


def _fused_kernel(x_ref, t_ref, noise_ref, W1_ref, b1_ref, W2_ref, b2_ref,
                  out_ref, mu_scr, h_scr, o_scr):
    D = x_ref.shape[1]
    t = t_ref[...]
    gamma = 1.0 - jnp.exp((2.0 * _LN_S1) * t)
    mu_coef = gamma * (1.0 - gamma)
    mu_scr[...] = gamma * x_ref[...] + mu_coef * noise_ref[...]

    h = jnp.dot(mu_scr[...], W1_ref[:D, :], preferred_element_type=jnp.float32)
    h = h + t * W1_ref[D:D + 1, :] + b1_ref[...]
    h_scr[...] = jnp.where(h >= 0, h, _LEAKY * h)

    o_scr[...] = jnp.dot(h_scr[...], W2_ref[...], preferred_element_type=jnp.float32)

    inv_gamma = 1.0 / gamma
    r = jnp.sqrt((1.0 - gamma) * inv_gamma)
    low_t = t < _TMIN
    w_row = jnp.exp((-2.0 * _LN_S1) * t)

    n_chunks = D // _C_BLK
    part = jnp.zeros((_B_BLK, _C_BLK), jnp.float32)
    for c in range(n_chunks):
        lo, hi = c * _C_BLK, (c + 1) * _C_BLK
        mu_eps = o_scr[:, lo:hi] + b2_ref[:, lo:hi]
        ln_sig = o_scr[:, D + lo:D + hi] + b2_ref[:, D + lo:D + hi]
        mu_c = mu_scr[:, lo:hi]
        mu_x = mu_c * inv_gamma - r * mu_eps
        sigma_x = r * jnp.exp(ln_sig)
        mu_x = jnp.where(low_t, 0.0, mu_x)
        sigma_x = jnp.where(low_t, 1.0, sigma_x)
        inv = jnp.minimum(1.0 / (sigma_x * jnp.float32(np.sqrt(2.0))), 1e30)
        ds = (2.0 / _K) * inv
        z1 = ((2.0 / _K - 1.0) - mu_x) * inv

        def body(k, carry):
            z, a0, a1 = carry
            e0 = jax.lax.erf(z)
            z = z + ds
            e1 = jax.lax.erf(z)
            z = z + ds
            return z, a0 + e0, a1 + e1

        zf, a0, a1 = jax.lax.fori_loop(
            0, (_K - 2) // 2, body,
            (z1, jnp.zeros_like(mu_x), jnp.zeros_like(mu_x)), unroll=3)
        s = a0 + a1
        e_last = jax.lax.erf(zf)
        pO = (-127.0 / 256.0) + (125.0 / 256.0) * e_last - (1.0 / 128.0) * s
        d = x_ref[:, lo:hi] - pO
        part = part + w_row * (d * d)

    out_ref[...] = jnp.sum(part, axis=0, keepdims=True).reshape(1, 1, _C_BLK)


def kernel(x, t, noise, W1, b1, W2, b2):
    B, D = x.shape
    H = W1.shape[1]
    nb = B // _B_BLK
    grid = (nb,)
    parts = pl.pallas_call(
        _fused_kernel,
        grid=grid,
        in_specs=[
            pl.BlockSpec((_B_BLK, D), lambda i: (i, 0)),
            pl.BlockSpec((_B_BLK, 1), lambda i: (i, 0)),
            pl.BlockSpec((_B_BLK, D), lambda i: (i, 0)),
            pl.BlockSpec((D + 1, H), lambda i: (0, 0)),
            pl.BlockSpec((1, H), lambda i: (0, 0)),
            pl.BlockSpec((H, 2 * D), lambda i: (0, 0)),
            pl.BlockSpec((1, 2 * D), lambda i: (0, 0)),
        ],
        out_specs=pl.BlockSpec((1, 1, _C_BLK), lambda i: (i, 0, 0)),
        out_shape=jax.ShapeDtypeStruct((nb, 1, _C_BLK), jnp.float32),
        scratch_shapes=[
            pltpu.VMEM((_B_BLK, D), jnp.float32),
            pltpu.VMEM((_B_BLK, H), jnp.float32),
            pltpu.VMEM((_B_BLK, 2 * D), jnp.float32),
        ],
        compiler_params=pltpu.CompilerParams(
            dimension_semantics=("parallel",),
            vmem_limit_bytes=100 * 1024 * 1024,
        ),
    )(x, t, noise, W1, b1.reshape(1, H), W2, b2.reshape(1, 2 * D))
    return (-_LN_S1 / (B * D)) * jnp.sum(parts)
